# Initial kernel scaffold; baseline (speedup 1.0000x reference)
import jax
import jax.numpy as jnp
from jax.experimental import pallas as pl


def kernel(p, q):
    raise NotImplementedError("write your pallas kernel here")


# speedup vs baseline: 1.4286x; 1.4286x over previous
import functools

import jax
import jax.numpy as jnp
from jax import lax
from jax.experimental import pallas as pl
from jax.experimental.pallas import tpu as pltpu
from jax.experimental.pallas import tpu_sc as plsc

TOPK = 50
EPSF = 1e-08
B = 128
V = 100000
CHUNK = 20000
NCHUNK = V // CHUNK
NVREG = CHUNK // 16
BUFCAP = 256
ROWS_PER_W = 4
NCORES = 2
NSUBCORES = 16
USLOT = 64


def _iota16():
    return lax.iota(jnp.int32, 16)


def _sort16_desc(x):
    k, _ = plsc.sort_key_val(x, x, descending=True)
    return k


def _merge_into_l(l_ref, s):
    c = s
    for j in range(4):
        a = l_ref[pl.ds(j * 16, 16)]
        rc = jnp.flip(c, 0)
        u = jnp.maximum(a, rc)
        lo = jnp.minimum(a, rc)
        l_ref[pl.ds(j * 16, 16)] = _sort16_desc(u)
        c = _sort16_desc(lo)


def _count(mask):
    return plsc.all_reduce_population_count(mask)[0]


def _sc_body(p_hbm, q_hbm, pu_hbm, qu_hbm,
             cb0, cb1, l_ref, buf_v,
             gt_v, gt_i, eq_i,
             sp_v, sp_i, sq_v, sq_i,
             qat, pat, gidx, prow, qrow,
             ns, fs,
             sem0, sem1, semg):
    wid = lax.axis_index("s") * NCORES + lax.axis_index("c")
    cbs = (cb0, cb1)
    sems = (sem0, sem1)

    def chunk_copy(a_hbm, row, c, par):
        return pltpu.make_async_copy(
            a_hbm.at[pl.ds(row * V + c * CHUNK, CHUNK)], cbs[par], sems[par])

    def compact():
        nb = ns[0]

        def mbody(j, _):
            rem = nb - j * 16
            x = buf_v[pl.ds(j * 16, 16)]
            x = jnp.where(_iota16() < rem, x, jnp.float32(-1.0))
            _merge_into_l(l_ref, _sort16_desc(x))
            return 0

        lax.fori_loop(0, (nb + 15) // 16, mbody, 0)
        ns[0] = 0
        fs[0] = l_ref[pl.ds(48, 16)][15]

    def phase1(a_hbm, row):
        for j in range(4):
            l_ref[pl.ds(j * 16, 16)] = jnp.full((16,), -1.0, jnp.float32)
        ns[0] = 0
        fs[0] = jnp.float32(-1.0)
        chunk_copy(a_hbm, row, 0, 0).start()
        for c in range(NCHUNK):
            par = c % 2
            chunk_copy(a_hbm, row, c, par).wait()
            if c + 1 < NCHUNK:
                chunk_copy(a_hbm, row, c + 1, (c + 1) % 2).start()
            cb = cbs[par]

            def vbody(i, _):
                v = cb[pl.ds(i * 16, 16)]
                t = jnp.full((16,), fs[0])
                m = v > t
                cnt = _count(m)

                def append():
                    n = ns[0]
                    plsc.store_compressed(buf_v.at[pl.ds(n, 16)], v, mask=m)
                    ns[0] = n + cnt

                pl.when(cnt > 0)(append)
                pl.when(ns[0] >= BUFCAP)(compact)
                return 0

            lax.fori_loop(0, NVREG, vbody, 0)
        compact()

    def phase2(a_hbm, row, dst_v, dst_i, do_qat):
        t49 = fs[1]
        ns[1] = 0
        ns[2] = 0
        chunk_copy(a_hbm, row, 0, 0).start()
        for c in range(NCHUNK):
            par = c % 2
            chunk_copy(a_hbm, row, c, par).wait()
            if c + 1 < NCHUNK:
                chunk_copy(a_hbm, row, c + 1, (c + 1) % 2).start()
            cb = cbs[par]

            def vbody(i, _):
                v = cb[pl.ds(i * 16, 16)]
                t = jnp.full((16,), t49)
                m_ge = v >= t
                c_ge = _count(m_ge)

                def slow():
                    m_gt = v > t
                    m_eq = m_ge & jnp.logical_not(m_gt)
                    idxv = jnp.full((16,), c * CHUNK + i * 16) + _iota16()
                    c_gt = _count(m_gt)
                    ng = ns[1]
                    plsc.store_compressed(gt_v.at[pl.ds(ng, 16)], v, mask=m_gt)
                    plsc.store_compressed(gt_i.at[pl.ds(ng, 16)], idxv,
                                          mask=m_gt)
                    ns[1] = ng + c_gt

                    def eq_append():
                        ne = ns[2]
                        plsc.store_compressed(eq_i.at[pl.ds(ne, 16)], idxv,
                                              mask=m_eq)
                        ns[2] = ne + _count(m_eq)

                    pl.when((ns[2] < TOPK) & (c_gt < c_ge))(eq_append)

                pl.when(c_ge > 0)(slow)
                return 0

            lax.fori_loop(0, NVREG, vbody, 0)
            if do_qat:
                base = c * CHUNK
                for j in range(4):
                    spi = sp_i[pl.ds(j * 16, 16)]
                    local = spi - base
                    inr = (local >= 0) & (local < CHUNK)
                    safe = jnp.where(inr, local, 0)
                    g = plsc.load_gather(cb, [safe], mask=inr)
                    cur = qat[pl.ds(j * 16, 16)]
                    qat[pl.ds(j * 16, 16)] = jnp.where(inr, g, cur)
        g_n = ns[1]
        for j in range(4):
            gl = j * 16 + _iota16()
            in_gt = gl < g_n
            valid = gl < TOPK
            gv = plsc.load_gather(gt_v, [jnp.where(in_gt, gl, 0)], mask=in_gt)
            gi = plsc.load_gather(gt_i, [jnp.where(in_gt, gl, 0)], mask=in_gt)
            ep = jnp.where(in_gt, 0, gl - g_n)
            ei = plsc.load_gather(eq_i, [ep], mask=valid & ~in_gt)
            val = jnp.where(in_gt, gv, jnp.full((16,), t49))
            idx = jnp.where(in_gt, gi, ei)
            dst_v[pl.ds(j * 16, 16)] = jnp.where(valid, val, jnp.float32(0.0))
            dst_i[pl.ds(j * 16, 16)] = jnp.where(valid, idx, -1)

    def row_body(r, _):
        row = wid * ROWS_PER_W + r
        phase1(p_hbm, row)
        fs[1] = l_ref[pl.ds(48, 16)][TOPK - 1 - 48]
        phase2(p_hbm, row, sp_v, sp_i, do_qat=False)
        for j in range(4):
            qat[pl.ds(j * 16, 16)] = jnp.zeros((16,), jnp.float32)
        phase1(q_hbm, row)
        fs[1] = l_ref[pl.ds(48, 16)][TOPK - 1 - 48]
        phase2(q_hbm, row, sq_v, sq_i, do_qat=True)
        rbase = row * V
        for j in range(4):
            sqi = sq_i[pl.ds(j * 16, 16)]
            gidx[pl.ds(j * 16, 16)] = jnp.where(sqi >= 0, rbase + sqi, rbase)
        pltpu.make_async_copy(p_hbm.at[gidx], pat, semg).start()
        pltpu.make_async_copy(p_hbm.at[gidx], pat, semg).wait()
        it16 = _iota16()
        for j in range(4):
            sqi = sq_i[pl.ds(j * 16, 16)]
            dup = jnp.zeros((16,), jnp.bool_)

            def dbody(k, dup):
                perm = jnp.where(it16 + k >= 16, it16 + k - 16, it16 + k)
                for jj in range(4):
                    spb = plsc.load_gather(sp_i, [jj * 16 + perm])
                    dup = dup | (sqi == spb)
                return dup

            dup = lax.fori_loop(0, 16, dbody, dup)
            keep = (sqi >= 0) & jnp.logical_not(dup)
            qv = sq_v[pl.ds(j * 16, 16)]
            pv = pat[pl.ds(j * 16, 16)]
            prow[pl.ds(USLOT + j * 16, 16)] = jnp.where(keep, pv, 0.0)
            qrow[pl.ds(USLOT + j * 16, 16)] = jnp.where(keep, qv, 0.0)
            prow[pl.ds(j * 16, 16)] = sp_v[pl.ds(j * 16, 16)]
            qrow[pl.ds(j * 16, 16)] = qat[pl.ds(j * 16, 16)]
        pltpu.sync_copy(prow, pu_hbm.at[row])
        pltpu.sync_copy(qrow, qu_hbm.at[row])
        return 0

    lax.fori_loop(0, ROWS_PER_W, row_body, 0)


def _jsd_tc(pu_ref, qu_ref, out_ref):
    eps = jnp.float32(EPSF)
    pu = pu_ref[...]
    qu = qu_ref[...]
    ps = jnp.sum(pu, axis=1, keepdims=True)
    qs = jnp.sum(qu, axis=1, keepdims=True)
    pn = pu / (ps + eps)
    qn = qu / (qs + eps)
    m = 0.5 * (pn + qn)
    ms = m + eps
    klp = jnp.sum((pn + eps) * jnp.log((pn + eps) / ms), axis=1,
                  keepdims=True)
    klq = jnp.sum((qn + eps) * jnp.log((qn + eps) / ms), axis=1,
                  keepdims=True)
    out_ref[...] = 0.5 * klp + 0.5 * klq


@jax.jit
def kernel(p, q):
    mesh = plsc.VectorSubcoreMesh(
        core_axis_name="c", subcore_axis_name="s",
        num_cores=NCORES, num_subcores=NSUBCORES)
    f32 = jnp.float32
    i32 = jnp.int32
    sc = pl.kernel(
        _sc_body,
        out_type=(
            jax.ShapeDtypeStruct((B, 2 * USLOT), f32),
            jax.ShapeDtypeStruct((B, 2 * USLOT), f32),
        ),
        mesh=mesh,
        scratch_types=[
            pltpu.VMEM((CHUNK,), f32),
            pltpu.VMEM((CHUNK,), f32),
            pltpu.VMEM((64,), f32),
            pltpu.VMEM((BUFCAP + 16,), f32),
            pltpu.VMEM((80,), f32),
            pltpu.VMEM((80,), i32),
            pltpu.VMEM((80,), i32),
            pltpu.VMEM((64,), f32),
            pltpu.VMEM((64,), i32),
            pltpu.VMEM((64,), f32),
            pltpu.VMEM((64,), i32),
            pltpu.VMEM((64,), f32),
            pltpu.VMEM((64,), f32),
            pltpu.VMEM((64,), i32),
            pltpu.VMEM((2 * USLOT,), f32),
            pltpu.VMEM((2 * USLOT,), f32),
            pltpu.SMEM((4,), i32),
            pltpu.SMEM((4,), f32),
            pltpu.SemaphoreType.DMA,
            pltpu.SemaphoreType.DMA,
            pltpu.SemaphoreType.DMA,
        ],
        compiler_params=pltpu.CompilerParams(
            use_tc_tiling_on_sc=False, needs_layout_passes=False),
    )
    pu, qu = sc(p.reshape(-1), q.reshape(-1))
    jsd = pl.pallas_call(
        _jsd_tc,
        out_shape=jax.ShapeDtypeStruct((B, 1), f32),
    )(pu, qu)
    return jsd.reshape(B)


# speedup vs baseline: 2.7215x; 1.9050x over previous
import functools

import jax
import jax.numpy as jnp
from jax import lax
from jax.experimental import pallas as pl
from jax.experimental.pallas import tpu as pltpu
from jax.experimental.pallas import tpu_sc as plsc

TOPK = 50
EPSF = 1e-08
B = 128
V = 100000
CHUNK = 20000
NCHUNK = V // CHUNK
NVREG = CHUNK // 16
BLK = 10
NBLK = NVREG // BLK
BUFCAP = 256
ROWS_PER_W = 4
NCORES = 2
NSUBCORES = 16
USLOT = 64


def _iota16():
    return lax.iota(jnp.int32, 16)


def _sort16_desc(x):
    k, _ = plsc.sort_key_val(x, x, descending=True)
    return k


def _merge_into_l(l_ref, s):
    c = s
    for j in range(4):
        a = l_ref[pl.ds(j * 16, 16)]
        rc = jnp.flip(c, 0)
        u = jnp.maximum(a, rc)
        lo = jnp.minimum(a, rc)
        l_ref[pl.ds(j * 16, 16)] = _sort16_desc(u)
        c = _sort16_desc(lo)


def _count(mask):
    return plsc.all_reduce_population_count(mask)[0]


def _sc_body(p_hbm, q_hbm, pu_hbm, qu_hbm,
             cb0, cb1, l_ref, buf_v,
             gt_v, gt_i, eq_i,
             sp_v, sp_i, sq_v, sq_i,
             qat, pat, gidx, prow, qrow,
             ns, fs,
             sem0, sem1, semg):
    wid = lax.axis_index("s") * NCORES + lax.axis_index("c")
    cbs = (cb0, cb1)
    sems = (sem0, sem1)

    def chunk_copy(a_hbm, row, c, par):
        return pltpu.make_async_copy(
            a_hbm.at[pl.ds(row * V + c * CHUNK, CHUNK)], cbs[par], sems[par])

    def compact():
        nb = ns[0]

        def mbody(j, _):
            rem = nb - j * 16
            x = buf_v[pl.ds(j * 16, 16)]
            x = jnp.where(_iota16() < rem, x, jnp.float32(-1.0))
            _merge_into_l(l_ref, _sort16_desc(x))
            return 0

        lax.fori_loop(0, (nb + 15) // 16, mbody, 0)
        ns[0] = 0
        fs[0] = l_ref[pl.ds(48, 16)][15]

    def phase1(a_hbm, row):
        for j in range(4):
            l_ref[pl.ds(j * 16, 16)] = jnp.full((16,), -1.0, jnp.float32)
        ns[0] = 0
        fs[0] = jnp.float32(-1.0)
        chunk_copy(a_hbm, row, 0, 0).start()
        for c in range(NCHUNK):
            par = c % 2
            chunk_copy(a_hbm, row, c, par).wait()
            if c + 1 < NCHUNK:
                chunk_copy(a_hbm, row, c + 1, (c + 1) % 2).start()
            cb = cbs[par]

            def bbody(b, _):
                t = jnp.full((16,), fs[0])
                acc = jnp.zeros((16,), jnp.bool_)
                for k in range(BLK):
                    v = cb[pl.ds((b * BLK + k) * 16, 16)]
                    acc = acc | (v > t)

                def slow():
                    for k in range(BLK):
                        v = cb[pl.ds((b * BLK + k) * 16, 16)]
                        m = v > t
                        cnt = _count(m)

                        def append(v=v, m=m, cnt=cnt):
                            n = ns[0]
                            plsc.store_compressed(
                                buf_v.at[pl.ds(n, 16)], v, mask=m)
                            ns[0] = n + cnt

                        pl.when(cnt > 0)(append)

                pl.when(_count(acc) > 0)(slow)
                pl.when(ns[0] >= BUFCAP)(compact)
                return 0

            lax.fori_loop(0, NBLK, bbody, 0)
        compact()

    def phase2(a_hbm, row, dst_v, dst_i, do_qat):
        t49 = fs[1]
        ns[1] = 0
        ns[2] = 0
        chunk_copy(a_hbm, row, 0, 0).start()
        for c in range(NCHUNK):
            par = c % 2
            chunk_copy(a_hbm, row, c, par).wait()
            if c + 1 < NCHUNK:
                chunk_copy(a_hbm, row, c + 1, (c + 1) % 2).start()
            cb = cbs[par]

            def bbody(b, _):
                t = jnp.full((16,), t49)
                acc = jnp.zeros((16,), jnp.bool_)
                for k in range(BLK):
                    v = cb[pl.ds((b * BLK + k) * 16, 16)]
                    acc = acc | (v >= t)

                def slow():
                    for k in range(BLK):
                        v = cb[pl.ds((b * BLK + k) * 16, 16)]
                        m_ge = v >= t
                        c_ge = _count(m_ge)

                        def hit(v=v, m_ge=m_ge, c_ge=c_ge, k=k):
                            m_gt = v > t
                            m_eq = m_ge & jnp.logical_not(m_gt)
                            idxv = (jnp.full((16,), c * CHUNK)
                                    + (b * BLK + k) * 16 + _iota16())
                            c_gt = _count(m_gt)
                            ng = ns[1]
                            plsc.store_compressed(
                                gt_v.at[pl.ds(ng, 16)], v, mask=m_gt)
                            plsc.store_compressed(
                                gt_i.at[pl.ds(ng, 16)], idxv, mask=m_gt)
                            ns[1] = ng + c_gt

                            def eq_append():
                                ne = ns[2]
                                plsc.store_compressed(
                                    eq_i.at[pl.ds(ne, 16)], idxv, mask=m_eq)
                                ns[2] = ne + _count(m_eq)

                            pl.when((ns[2] < TOPK) & (c_gt < c_ge))(eq_append)

                        pl.when(c_ge > 0)(hit)

                pl.when(_count(acc) > 0)(slow)
                return 0

            lax.fori_loop(0, NBLK, bbody, 0)
            if do_qat:
                base = c * CHUNK
                for j in range(4):
                    spi = sp_i[pl.ds(j * 16, 16)]
                    local = spi - base
                    inr = (local >= 0) & (local < CHUNK)
                    safe = jnp.where(inr, local, 0)
                    g = plsc.load_gather(cb, [safe], mask=inr)
                    cur = qat[pl.ds(j * 16, 16)]
                    qat[pl.ds(j * 16, 16)] = jnp.where(inr, g, cur)
        g_n = ns[1]
        for j in range(4):
            gl = j * 16 + _iota16()
            in_gt = gl < g_n
            valid = gl < TOPK
            gv = plsc.load_gather(gt_v, [jnp.where(in_gt, gl, 0)], mask=in_gt)
            gi = plsc.load_gather(gt_i, [jnp.where(in_gt, gl, 0)], mask=in_gt)
            ep = jnp.where(in_gt, 0, gl - g_n)
            ei = plsc.load_gather(eq_i, [ep], mask=valid & ~in_gt)
            val = jnp.where(in_gt, gv, jnp.full((16,), t49))
            idx = jnp.where(in_gt, gi, ei)
            dst_v[pl.ds(j * 16, 16)] = jnp.where(valid, val, jnp.float32(0.0))
            dst_i[pl.ds(j * 16, 16)] = jnp.where(valid, idx, -1)

    def row_body(r, _):
        row = wid * ROWS_PER_W + r
        phase1(p_hbm, row)
        fs[1] = l_ref[pl.ds(48, 16)][TOPK - 1 - 48]
        phase2(p_hbm, row, sp_v, sp_i, do_qat=False)
        for j in range(4):
            qat[pl.ds(j * 16, 16)] = jnp.zeros((16,), jnp.float32)
        phase1(q_hbm, row)
        fs[1] = l_ref[pl.ds(48, 16)][TOPK - 1 - 48]
        phase2(q_hbm, row, sq_v, sq_i, do_qat=True)
        rbase = row * V
        for j in range(4):
            sqi = sq_i[pl.ds(j * 16, 16)]
            gidx[pl.ds(j * 16, 16)] = jnp.where(sqi >= 0, rbase + sqi, rbase)
        pltpu.make_async_copy(p_hbm.at[gidx], pat, semg).start()
        pltpu.make_async_copy(p_hbm.at[gidx], pat, semg).wait()
        it16 = _iota16()
        for j in range(4):
            sqi = sq_i[pl.ds(j * 16, 16)]
            dup = jnp.zeros((16,), jnp.bool_)

            def dbody(k, dup):
                perm = jnp.where(it16 + k >= 16, it16 + k - 16, it16 + k)
                for jj in range(4):
                    spb = plsc.load_gather(sp_i, [jj * 16 + perm])
                    dup = dup | (sqi == spb)
                return dup

            dup = lax.fori_loop(0, 16, dbody, dup)
            keep = (sqi >= 0) & jnp.logical_not(dup)
            qv = sq_v[pl.ds(j * 16, 16)]
            pv = pat[pl.ds(j * 16, 16)]
            prow[pl.ds(USLOT + j * 16, 16)] = jnp.where(keep, pv, 0.0)
            qrow[pl.ds(USLOT + j * 16, 16)] = jnp.where(keep, qv, 0.0)
            prow[pl.ds(j * 16, 16)] = sp_v[pl.ds(j * 16, 16)]
            qrow[pl.ds(j * 16, 16)] = qat[pl.ds(j * 16, 16)]
        pltpu.sync_copy(prow, pu_hbm.at[row])
        pltpu.sync_copy(qrow, qu_hbm.at[row])
        return 0

    lax.fori_loop(0, ROWS_PER_W, row_body, 0)


def _jsd_tc(pu_ref, qu_ref, out_ref):
    eps = jnp.float32(EPSF)
    pu = pu_ref[...]
    qu = qu_ref[...]
    ps = jnp.sum(pu, axis=1, keepdims=True)
    qs = jnp.sum(qu, axis=1, keepdims=True)
    pn = pu / (ps + eps)
    qn = qu / (qs + eps)
    m = 0.5 * (pn + qn)
    ms = m + eps
    klp = jnp.sum((pn + eps) * jnp.log((pn + eps) / ms), axis=1,
                  keepdims=True)
    klq = jnp.sum((qn + eps) * jnp.log((qn + eps) / ms), axis=1,
                  keepdims=True)
    out_ref[...] = 0.5 * klp + 0.5 * klq


@jax.jit
def kernel(p, q):
    mesh = plsc.VectorSubcoreMesh(
        core_axis_name="c", subcore_axis_name="s",
        num_cores=NCORES, num_subcores=NSUBCORES)
    f32 = jnp.float32
    i32 = jnp.int32
    sc = pl.kernel(
        _sc_body,
        out_type=(
            jax.ShapeDtypeStruct((B, 2 * USLOT), f32),
            jax.ShapeDtypeStruct((B, 2 * USLOT), f32),
        ),
        mesh=mesh,
        scratch_types=[
            pltpu.VMEM((CHUNK,), f32),
            pltpu.VMEM((CHUNK,), f32),
            pltpu.VMEM((64,), f32),
            pltpu.VMEM((BUFCAP + BLK * 16,), f32),
            pltpu.VMEM((80,), f32),
            pltpu.VMEM((80,), i32),
            pltpu.VMEM((80,), i32),
            pltpu.VMEM((64,), f32),
            pltpu.VMEM((64,), i32),
            pltpu.VMEM((64,), f32),
            pltpu.VMEM((64,), i32),
            pltpu.VMEM((64,), f32),
            pltpu.VMEM((64,), f32),
            pltpu.VMEM((64,), i32),
            pltpu.VMEM((2 * USLOT,), f32),
            pltpu.VMEM((2 * USLOT,), f32),
            pltpu.SMEM((4,), i32),
            pltpu.SMEM((4,), f32),
            pltpu.SemaphoreType.DMA,
            pltpu.SemaphoreType.DMA,
            pltpu.SemaphoreType.DMA,
        ],
        compiler_params=pltpu.CompilerParams(
            use_tc_tiling_on_sc=False, needs_layout_passes=False),
    )
    pu, qu = sc(p.reshape(-1), q.reshape(-1))
    jsd = pl.pallas_call(
        _jsd_tc,
        out_shape=jax.ShapeDtypeStruct((B, 1), f32),
    )(pu, qu)
    return jsd.reshape(B)


# speedup vs baseline: 3.0811x; 1.1321x over previous
import functools

import jax
import jax.numpy as jnp
from jax import lax
from jax.experimental import pallas as pl
from jax.experimental.pallas import tpu as pltpu
from jax.experimental.pallas import tpu_sc as plsc

TOPK = 50
EPSF = 1e-08
B = 128
V = 100000
CHUNK = 20000
NCHUNK = V // CHUNK
NVREG = CHUNK // 16
BLK = 10
NBLK = NVREG // BLK
BUFCAP = 256
ROWS_PER_W = 4
NCORES = 2
NSUBCORES = 16
USLOT = 64


def _iota16():
    return lax.iota(jnp.int32, 16)


def _sort16_desc(x):
    k, _ = plsc.sort_key_val(x, x, descending=True)
    return k


def _merge_into_l(l_ref, s):
    c = s
    for j in range(4):
        a = l_ref[pl.ds(j * 16, 16)]
        rc = jnp.flip(c, 0)
        u = jnp.maximum(a, rc)
        lo = jnp.minimum(a, rc)
        l_ref[pl.ds(j * 16, 16)] = _sort16_desc(u)
        c = _sort16_desc(lo)


def _count(mask):
    return plsc.all_reduce_population_count(mask)[0]


def _sc_body(p_hbm, q_hbm, pu_hbm, qu_hbm,
             cb0, cb1, l_ref, buf_v, buf_i, arch_v, arch_i,
             gt_v, gt_i, eq_i,
             sp_v, sp_i, sq_v, sq_i,
             qat, pat, gidx, prow, qrow,
             ns, fs,
             sem0, sem1, semg):
    wid = lax.axis_index("s") * NCORES + lax.axis_index("c")
    cbs = (cb0, cb1)
    sems = (sem0, sem1)

    def chunk_copy(a_hbm, row, c, par):
        return pltpu.make_async_copy(
            a_hbm.at[pl.ds(row * V + c * CHUNK, CHUNK)], cbs[par], sems[par])

    def compact():
        nb = ns[0]

        def mbody(j, _):
            rem = nb - j * 16
            x = buf_v[pl.ds(j * 16, 16)]
            x = jnp.where(_iota16() < rem, x, jnp.float32(-1.0))
            _merge_into_l(l_ref, _sort16_desc(x))
            return 0

        lax.fori_loop(0, (nb + 15) // 16, mbody, 0)
        lv = l_ref[pl.ds(48, 16)]
        fs[0] = lv[15]
        fs[1] = lv[TOPK - 1 - 48]
        tv = jnp.full((16,), fs[0])
        na = ns[3]
        ns[3] = 0
        ns[4] = 0

        def prune_vec(v, iv, rem):
            mval = _iota16() < rem
            m_gt = mval & (v > tv)
            m_eq = mval & (v == tv)
            cs = plsc.cumsum(m_eq.astype(jnp.int32))
            keep_eq = m_eq & ((ns[4] + cs) <= TOPK)
            keep = m_gt | keep_eq
            nn = ns[3]
            plsc.store_compressed(arch_v.at[pl.ds(nn, 16)], v, mask=keep)
            plsc.store_compressed(arch_i.at[pl.ds(nn, 16)], iv, mask=keep)
            ns[3] = nn + _count(keep)
            ns[4] = ns[4] + _count(keep_eq)

        def abody(j, _):
            prune_vec(arch_v[pl.ds(j * 16, 16)], arch_i[pl.ds(j * 16, 16)],
                      na - j * 16)
            return 0

        lax.fori_loop(0, (na + 15) // 16, abody, 0)

        def pbody(j, _):
            prune_vec(buf_v[pl.ds(j * 16, 16)], buf_i[pl.ds(j * 16, 16)],
                      nb - j * 16)
            return 0

        lax.fori_loop(0, (nb + 15) // 16, pbody, 0)
        ns[0] = 0

    def scan_row(a_hbm, row, dst_v, dst_i, do_qat):
        for j in range(4):
            l_ref[pl.ds(j * 16, 16)] = jnp.full((16,), -1.0, jnp.float32)
        ns[0] = 0
        ns[3] = 0
        fs[0] = jnp.float32(-1.0)
        chunk_copy(a_hbm, row, 0, 0).start()
        for c in range(NCHUNK):
            par = c % 2
            chunk_copy(a_hbm, row, c, par).wait()
            if c + 1 < NCHUNK:
                chunk_copy(a_hbm, row, c + 1, (c + 1) % 2).start()
            cb = cbs[par]

            def bbody(b, _):
                t = jnp.full((16,), fs[0])
                acc = jnp.zeros((16,), jnp.bool_)
                for k in range(BLK):
                    v = cb[pl.ds((b * BLK + k) * 16, 16)]
                    acc = acc | (v >= t)

                def slow():
                    for k in range(BLK):
                        v = cb[pl.ds((b * BLK + k) * 16, 16)]
                        m = v >= t
                        cnt = _count(m)

                        def append(v=v, m=m, cnt=cnt, k=k):
                            n = ns[0]
                            idxv = (jnp.full((16,), c * CHUNK)
                                    + (b * BLK + k) * 16 + _iota16())
                            plsc.store_compressed(
                                buf_v.at[pl.ds(n, 16)], v, mask=m)
                            plsc.store_compressed(
                                buf_i.at[pl.ds(n, 16)], idxv, mask=m)
                            ns[0] = n + cnt

                        pl.when(cnt > 0)(append)

                pl.when(_count(acc) > 0)(slow)
                pl.when(ns[0] >= BUFCAP)(compact)
                return 0

            lax.fori_loop(0, NBLK, bbody, 0)
            if do_qat:
                base = c * CHUNK
                for j in range(4):
                    spi = sp_i[pl.ds(j * 16, 16)]
                    local = spi - base
                    inr = (local >= 0) & (local < CHUNK)
                    safe = jnp.where(inr, local, 0)
                    g = plsc.load_gather(cb, [safe], mask=inr)
                    cur = qat[pl.ds(j * 16, 16)]
                    qat[pl.ds(j * 16, 16)] = jnp.where(inr, g, cur)
        compact()
        t49 = fs[1]
        t49v = jnp.full((16,), t49)
        na = ns[3]
        ns[1] = 0
        ns[2] = 0

        def selbody(j, _):
            mval = _iota16() < (na - j * 16)
            v = arch_v[pl.ds(j * 16, 16)]
            iv = arch_i[pl.ds(j * 16, 16)]
            m_gt = mval & (v > t49v)
            m_eq = mval & (v == t49v)
            ng = ns[1]
            plsc.store_compressed(gt_v.at[pl.ds(ng, 16)], v, mask=m_gt)
            plsc.store_compressed(gt_i.at[pl.ds(ng, 16)], iv, mask=m_gt)
            ns[1] = ng + _count(m_gt)
            ne = ns[2]
            plsc.store_compressed(eq_i.at[pl.ds(ne, 16)], iv, mask=m_eq)
            ns[2] = ne + _count(m_eq)
            return 0

        lax.fori_loop(0, (na + 15) // 16, selbody, 0)
        g_n = ns[1]
        for j in range(4):
            gl = j * 16 + _iota16()
            in_gt = gl < g_n
            valid = gl < TOPK
            gv = plsc.load_gather(gt_v, [jnp.where(in_gt, gl, 0)], mask=in_gt)
            gi = plsc.load_gather(gt_i, [jnp.where(in_gt, gl, 0)], mask=in_gt)
            ep = jnp.where(in_gt, 0, gl - g_n)
            ei = plsc.load_gather(eq_i, [ep], mask=valid & ~in_gt)
            val = jnp.where(in_gt, gv, jnp.full((16,), t49))
            idx = jnp.where(in_gt, gi, ei)
            dst_v[pl.ds(j * 16, 16)] = jnp.where(valid, val, jnp.float32(0.0))
            dst_i[pl.ds(j * 16, 16)] = jnp.where(valid, idx, -1)

    def row_body(r, _):
        row = wid * ROWS_PER_W + r
        scan_row(p_hbm, row, sp_v, sp_i, do_qat=False)
        for j in range(4):
            qat[pl.ds(j * 16, 16)] = jnp.zeros((16,), jnp.float32)
        scan_row(q_hbm, row, sq_v, sq_i, do_qat=True)
        rbase = row * V
        for j in range(4):
            sqi = sq_i[pl.ds(j * 16, 16)]
            gidx[pl.ds(j * 16, 16)] = jnp.where(sqi >= 0, rbase + sqi, rbase)
        pltpu.make_async_copy(p_hbm.at[gidx], pat, semg).start()
        pltpu.make_async_copy(p_hbm.at[gidx], pat, semg).wait()
        it16 = _iota16()
        for j in range(4):
            sqi = sq_i[pl.ds(j * 16, 16)]
            dup = jnp.zeros((16,), jnp.bool_)

            def dbody(k, dup):
                perm = jnp.where(it16 + k >= 16, it16 + k - 16, it16 + k)
                for jj in range(4):
                    spb = plsc.load_gather(sp_i, [jj * 16 + perm])
                    dup = dup | (sqi == spb)
                return dup

            dup = lax.fori_loop(0, 16, dbody, dup)
            keep = (sqi >= 0) & jnp.logical_not(dup)
            qv = sq_v[pl.ds(j * 16, 16)]
            pv = pat[pl.ds(j * 16, 16)]
            prow[pl.ds(USLOT + j * 16, 16)] = jnp.where(keep, pv, 0.0)
            qrow[pl.ds(USLOT + j * 16, 16)] = jnp.where(keep, qv, 0.0)
            prow[pl.ds(j * 16, 16)] = sp_v[pl.ds(j * 16, 16)]
            qrow[pl.ds(j * 16, 16)] = qat[pl.ds(j * 16, 16)]
        pltpu.sync_copy(prow, pu_hbm.at[row])
        pltpu.sync_copy(qrow, qu_hbm.at[row])
        return 0

    lax.fori_loop(0, ROWS_PER_W, row_body, 0)


def _jsd_tc(pu_ref, qu_ref, out_ref):
    eps = jnp.float32(EPSF)
    pu = pu_ref[...]
    qu = qu_ref[...]
    ps = jnp.sum(pu, axis=1, keepdims=True)
    qs = jnp.sum(qu, axis=1, keepdims=True)
    pn = pu / (ps + eps)
    qn = qu / (qs + eps)
    m = 0.5 * (pn + qn)
    ms = m + eps
    klp = jnp.sum((pn + eps) * jnp.log((pn + eps) / ms), axis=1,
                  keepdims=True)
    klq = jnp.sum((qn + eps) * jnp.log((qn + eps) / ms), axis=1,
                  keepdims=True)
    out_ref[...] = 0.5 * klp + 0.5 * klq


@jax.jit
def kernel(p, q):
    mesh = plsc.VectorSubcoreMesh(
        core_axis_name="c", subcore_axis_name="s",
        num_cores=NCORES, num_subcores=NSUBCORES)
    f32 = jnp.float32
    i32 = jnp.int32
    sc = pl.kernel(
        _sc_body,
        out_type=(
            jax.ShapeDtypeStruct((B, 2 * USLOT), f32),
            jax.ShapeDtypeStruct((B, 2 * USLOT), f32),
        ),
        mesh=mesh,
        scratch_types=[
            pltpu.VMEM((CHUNK,), f32),
            pltpu.VMEM((CHUNK,), f32),
            pltpu.VMEM((64,), f32),
            pltpu.VMEM((BUFCAP + BLK * 16,), f32),
            pltpu.VMEM((BUFCAP + BLK * 16,), i32),
            pltpu.VMEM((144,), f32),
            pltpu.VMEM((144,), i32),
            pltpu.VMEM((80,), f32),
            pltpu.VMEM((80,), i32),
            pltpu.VMEM((80,), i32),
            pltpu.VMEM((64,), f32),
            pltpu.VMEM((64,), i32),
            pltpu.VMEM((64,), f32),
            pltpu.VMEM((64,), i32),
            pltpu.VMEM((64,), f32),
            pltpu.VMEM((64,), f32),
            pltpu.VMEM((64,), i32),
            pltpu.VMEM((2 * USLOT,), f32),
            pltpu.VMEM((2 * USLOT,), f32),
            pltpu.SMEM((8,), i32),
            pltpu.SMEM((4,), f32),
            pltpu.SemaphoreType.DMA,
            pltpu.SemaphoreType.DMA,
            pltpu.SemaphoreType.DMA,
        ],
        compiler_params=pltpu.CompilerParams(
            use_tc_tiling_on_sc=False, needs_layout_passes=False),
    )
    pu, qu = sc(p.reshape(-1), q.reshape(-1))
    jsd = pl.pallas_call(
        _jsd_tc,
        out_shape=jax.ShapeDtypeStruct((B, 1), f32),
    )(pu, qu)
    return jsd.reshape(B)


# speedup vs baseline: 4.3818x; 1.4222x over previous
import functools

import jax
import jax.numpy as jnp
from jax import lax
from jax.experimental import pallas as pl
from jax.experimental.pallas import tpu as pltpu
from jax.experimental.pallas import tpu_sc as plsc

TOPK = 50
EPSF = 1e-08
B = 128
V = 100000
CHUNK = 20000
NCHUNK = V // CHUNK
NVREG = CHUNK // 16
BLK = 10
NBLK = NVREG // BLK
BUFCAP = 256
ROWS_PER_W = 4
NCORES = 2
NSUBCORES = 16
USLOT = 64


def _iota16():
    return lax.iota(jnp.int32, 16)


def _sort16_desc(x):
    k, _ = plsc.sort_key_val(x, x, descending=True)
    return k


def _merge_into_l(l_ref, s):
    c = s
    for j in range(4):
        a = l_ref[pl.ds(j * 16, 16)]
        rc = jnp.flip(c, 0)
        u = jnp.maximum(a, rc)
        lo = jnp.minimum(a, rc)
        l_ref[pl.ds(j * 16, 16)] = _sort16_desc(u)
        c = _sort16_desc(lo)


def _count(mask):
    return plsc.all_reduce_population_count(mask)[0]


def _sc_body(p_hbm, q_hbm, pu_hbm, qu_hbm,
             cb0, cb1, l_ref, buf_v, buf_i, arch_v, arch_i,
             gt_v, gt_i, eq_i,
             sp_v, sp_i, sq_v, sq_i,
             qat, pat, gidx, prow, qrow,
             ns, fs,
             sem0, sem1, semg):
    wid = lax.axis_index("s") * NCORES + lax.axis_index("c")
    cbs = (cb0, cb1)
    sems = (sem0, sem1)

    def chunk_copy(a_hbm, row, c, par):
        return pltpu.make_async_copy(
            a_hbm.at[pl.ds(row * V + c * CHUNK, CHUNK)], cbs[par], sems[par])

    def compact():
        nb = ns[0]

        def mbody(j, _):
            rem = nb - j * 16
            x = buf_v[pl.ds(j * 16, 16)]
            x = jnp.where(_iota16() < rem, x, jnp.float32(-1.0))
            _merge_into_l(l_ref, _sort16_desc(x))
            return 0

        lax.fori_loop(0, (nb + 15) // 16, mbody, 0)
        lv = l_ref[pl.ds(48, 16)]
        fs[0] = lv[15]
        fs[1] = lv[TOPK - 1 - 48]
        tv = jnp.full((16,), fs[0])
        na = ns[3]
        ns[3] = 0
        ns[4] = 0

        def prune_vec(v, iv, rem):
            mval = _iota16() < rem
            m_gt = mval & (v > tv)
            m_eq = mval & (v == tv)
            cs = plsc.cumsum(m_eq.astype(jnp.int32))
            keep_eq = m_eq & ((ns[4] + cs) <= TOPK)
            keep = m_gt | keep_eq
            nn = ns[3]
            plsc.store_compressed(arch_v.at[pl.ds(nn, 16)], v, mask=keep)
            plsc.store_compressed(arch_i.at[pl.ds(nn, 16)], iv, mask=keep)
            ns[3] = nn + _count(keep)
            ns[4] = ns[4] + _count(keep_eq)

        def abody(j, _):
            prune_vec(arch_v[pl.ds(j * 16, 16)], arch_i[pl.ds(j * 16, 16)],
                      na - j * 16)
            return 0

        lax.fori_loop(0, (na + 15) // 16, abody, 0)

        def pbody(j, _):
            prune_vec(buf_v[pl.ds(j * 16, 16)], buf_i[pl.ds(j * 16, 16)],
                      nb - j * 16)
            return 0

        lax.fori_loop(0, (nb + 15) // 16, pbody, 0)
        ns[0] = 0

    def scan_row(a_hbm, row, dst_v, dst_i, do_qat):
        for j in range(4):
            l_ref[pl.ds(j * 16, 16)] = jnp.full((16,), -1.0, jnp.float32)
        ns[0] = 0
        ns[3] = 0
        fs[0] = jnp.float32(-1.0)
        chunk_copy(a_hbm, row, 0, 0).start()
        for c in range(NCHUNK):
            par = c % 2
            chunk_copy(a_hbm, row, c, par).wait()
            if c + 1 < NCHUNK:
                chunk_copy(a_hbm, row, c + 1, (c + 1) % 2).start()
            cb = cbs[par]

            def append_vreg(v, m, idxv):
                n = ns[0]
                plsc.store_compressed(buf_v.at[pl.ds(n, 16)], v, mask=m)
                plsc.store_compressed(buf_i.at[pl.ds(n, 16)], idxv, mask=m)
                ns[0] = n + _count(m)

            if c == 0:
                def vbody(i, _):
                    v = cb[pl.ds(i * 16, 16)]
                    m = v >= jnp.full((16,), fs[0])
                    append_vreg(v, m, jnp.full((16,), i * 16) + _iota16())
                    pl.when(ns[0] >= BUFCAP)(compact)
                    return 0

                lax.fori_loop(0, NVREG, vbody, 0)
            else:
                def bbody(b, _):
                    t = jnp.full((16,), fs[0])
                    acc = jnp.zeros((16,), jnp.bool_)
                    for k in range(BLK):
                        v = cb[pl.ds((b * BLK + k) * 16, 16)]
                        acc = acc | (v >= t)

                    def slow():
                        for k in range(BLK):
                            v = cb[pl.ds((b * BLK + k) * 16, 16)]
                            idxv = (jnp.full((16,), c * CHUNK)
                                    + (b * BLK + k) * 16 + _iota16())
                            append_vreg(v, v >= t, idxv)

                    pl.when(_count(acc) > 0)(slow)
                    pl.when(ns[0] >= BUFCAP)(compact)
                    return 0

                lax.fori_loop(0, NBLK, bbody, 0)
            if do_qat:
                base = c * CHUNK
                for j in range(4):
                    spi = sp_i[pl.ds(j * 16, 16)]
                    local = spi - base
                    inr = (local >= 0) & (local < CHUNK)
                    safe = jnp.where(inr, local, 0)
                    g = plsc.load_gather(cb, [safe], mask=inr)
                    cur = qat[pl.ds(j * 16, 16)]
                    qat[pl.ds(j * 16, 16)] = jnp.where(inr, g, cur)
        compact()
        t49 = fs[1]
        t49v = jnp.full((16,), t49)
        na = ns[3]
        ns[1] = 0
        ns[2] = 0

        def selbody(j, _):
            mval = _iota16() < (na - j * 16)
            v = arch_v[pl.ds(j * 16, 16)]
            iv = arch_i[pl.ds(j * 16, 16)]
            m_gt = mval & (v > t49v)
            m_eq = mval & (v == t49v)
            ng = ns[1]
            plsc.store_compressed(gt_v.at[pl.ds(ng, 16)], v, mask=m_gt)
            plsc.store_compressed(gt_i.at[pl.ds(ng, 16)], iv, mask=m_gt)
            ns[1] = ng + _count(m_gt)
            ne = ns[2]
            plsc.store_compressed(eq_i.at[pl.ds(ne, 16)], iv, mask=m_eq)
            ns[2] = ne + _count(m_eq)
            return 0

        lax.fori_loop(0, (na + 15) // 16, selbody, 0)
        g_n = ns[1]
        for j in range(4):
            gl = j * 16 + _iota16()
            in_gt = gl < g_n
            valid = gl < TOPK
            gv = plsc.load_gather(gt_v, [jnp.where(in_gt, gl, 0)], mask=in_gt)
            gi = plsc.load_gather(gt_i, [jnp.where(in_gt, gl, 0)], mask=in_gt)
            ep = jnp.where(in_gt, 0, gl - g_n)
            ei = plsc.load_gather(eq_i, [ep], mask=valid & ~in_gt)
            val = jnp.where(in_gt, gv, jnp.full((16,), t49))
            idx = jnp.where(in_gt, gi, ei)
            dst_v[pl.ds(j * 16, 16)] = jnp.where(valid, val, jnp.float32(0.0))
            dst_i[pl.ds(j * 16, 16)] = jnp.where(valid, idx, -1)

    def row_body(r, _):
        row = wid * ROWS_PER_W + r
        scan_row(p_hbm, row, sp_v, sp_i, do_qat=False)
        for j in range(4):
            qat[pl.ds(j * 16, 16)] = jnp.zeros((16,), jnp.float32)
        scan_row(q_hbm, row, sq_v, sq_i, do_qat=True)
        rbase = row * V
        for j in range(4):
            sqi = sq_i[pl.ds(j * 16, 16)]
            gidx[pl.ds(j * 16, 16)] = jnp.where(sqi >= 0, rbase + sqi, rbase)
        pltpu.make_async_copy(p_hbm.at[gidx], pat, semg).start()
        pltpu.make_async_copy(p_hbm.at[gidx], pat, semg).wait()
        it16 = _iota16()
        for j in range(4):
            sqi = sq_i[pl.ds(j * 16, 16)]
            dup = jnp.zeros((16,), jnp.bool_)

            def dbody(k, dup):
                perm = jnp.where(it16 + k >= 16, it16 + k - 16, it16 + k)
                for jj in range(4):
                    spb = plsc.load_gather(sp_i, [jj * 16 + perm])
                    dup = dup | (sqi == spb)
                return dup

            dup = lax.fori_loop(0, 16, dbody, dup)
            keep = (sqi >= 0) & jnp.logical_not(dup)
            qv = sq_v[pl.ds(j * 16, 16)]
            pv = pat[pl.ds(j * 16, 16)]
            prow[pl.ds(USLOT + j * 16, 16)] = jnp.where(keep, pv, 0.0)
            qrow[pl.ds(USLOT + j * 16, 16)] = jnp.where(keep, qv, 0.0)
            prow[pl.ds(j * 16, 16)] = sp_v[pl.ds(j * 16, 16)]
            qrow[pl.ds(j * 16, 16)] = qat[pl.ds(j * 16, 16)]
        pltpu.sync_copy(prow, pu_hbm.at[row])
        pltpu.sync_copy(qrow, qu_hbm.at[row])
        return 0

    lax.fori_loop(0, ROWS_PER_W, row_body, 0)


def _jsd_tc(pu_ref, qu_ref, out_ref):
    eps = jnp.float32(EPSF)
    pu = pu_ref[...]
    qu = qu_ref[...]
    ps = jnp.sum(pu, axis=1, keepdims=True)
    qs = jnp.sum(qu, axis=1, keepdims=True)
    pn = pu / (ps + eps)
    qn = qu / (qs + eps)
    m = 0.5 * (pn + qn)
    ms = m + eps
    klp = jnp.sum((pn + eps) * jnp.log((pn + eps) / ms), axis=1,
                  keepdims=True)
    klq = jnp.sum((qn + eps) * jnp.log((qn + eps) / ms), axis=1,
                  keepdims=True)
    out_ref[...] = 0.5 * klp + 0.5 * klq


@jax.jit
def kernel(p, q):
    mesh = plsc.VectorSubcoreMesh(
        core_axis_name="c", subcore_axis_name="s",
        num_cores=NCORES, num_subcores=NSUBCORES)
    f32 = jnp.float32
    i32 = jnp.int32
    sc = pl.kernel(
        _sc_body,
        out_type=(
            jax.ShapeDtypeStruct((B, 2 * USLOT), f32),
            jax.ShapeDtypeStruct((B, 2 * USLOT), f32),
        ),
        mesh=mesh,
        scratch_types=[
            pltpu.VMEM((CHUNK,), f32),
            pltpu.VMEM((CHUNK,), f32),
            pltpu.VMEM((64,), f32),
            pltpu.VMEM((BUFCAP + BLK * 16,), f32),
            pltpu.VMEM((BUFCAP + BLK * 16,), i32),
            pltpu.VMEM((144,), f32),
            pltpu.VMEM((144,), i32),
            pltpu.VMEM((80,), f32),
            pltpu.VMEM((80,), i32),
            pltpu.VMEM((80,), i32),
            pltpu.VMEM((64,), f32),
            pltpu.VMEM((64,), i32),
            pltpu.VMEM((64,), f32),
            pltpu.VMEM((64,), i32),
            pltpu.VMEM((64,), f32),
            pltpu.VMEM((64,), f32),
            pltpu.VMEM((64,), i32),
            pltpu.VMEM((2 * USLOT,), f32),
            pltpu.VMEM((2 * USLOT,), f32),
            pltpu.SMEM((8,), i32),
            pltpu.SMEM((4,), f32),
            pltpu.SemaphoreType.DMA,
            pltpu.SemaphoreType.DMA,
            pltpu.SemaphoreType.DMA,
        ],
        compiler_params=pltpu.CompilerParams(
            use_tc_tiling_on_sc=False, needs_layout_passes=False),
    )
    pu, qu = sc(p.reshape(-1), q.reshape(-1))
    jsd = pl.pallas_call(
        _jsd_tc,
        out_shape=jax.ShapeDtypeStruct((B, 1), f32),
    )(pu, qu)
    return jsd.reshape(B)


# speedup vs baseline: 4.5917x; 1.0479x over previous
import functools

import jax
import jax.numpy as jnp
from jax import lax
from jax.experimental import pallas as pl
from jax.experimental.pallas import tpu as pltpu
from jax.experimental.pallas import tpu_sc as plsc

TOPK = 50
EPSF = 1e-08
B = 128
V = 100000
CHUNK = 20000
NCHUNK = V // CHUNK
NVREG = CHUNK // 16
BLK = 10
NBLK = NVREG // BLK
SEG = 50
BUFCAP = 256
BUFSZ = BUFCAP + SEG * 16 + 48
ROWS_PER_W = 4
NCORES = 2
NSUBCORES = 16
USLOT = 64


def _iota16():
    return lax.iota(jnp.int32, 16)


def _sort16_desc(x):
    k, _ = plsc.sort_key_val(x, x, descending=True)
    return k


def _merge_into_l(l_ref, s):
    c = s
    for j in range(4):
        a = l_ref[pl.ds(j * 16, 16)]
        rc = jnp.flip(c, 0)
        u = jnp.maximum(a, rc)
        lo = jnp.minimum(a, rc)
        l_ref[pl.ds(j * 16, 16)] = _sort16_desc(u)
        c = _sort16_desc(lo)


def _count(mask):
    return plsc.all_reduce_population_count(mask)[0]


def _sc_body(p_hbm, q_hbm, pu_hbm, qu_hbm,
             cb0, cb1, l_ref, buf_v, buf_i, arch_v, arch_i,
             gt_v, gt_i, eq_i,
             sp_v, sp_i, sq_v, sq_i,
             qat, pat, gidx, prow, qrow,
             ns, fs,
             sem0, sem1, semg):
    wid = lax.axis_index("s") * NCORES + lax.axis_index("c")
    cbs = (cb0, cb1)
    sems = (sem0, sem1)

    def chunk_copy(a_hbm, row, c, par):
        return pltpu.make_async_copy(
            a_hbm.at[pl.ds(row * V + c * CHUNK, CHUNK)], cbs[par], sems[par])

    def compact():
        nb = ns[0]

        def mbody(j, _):
            rem = nb - j * 16
            x = buf_v[pl.ds(j * 16, 16)]
            x = jnp.where(_iota16() < rem, x, jnp.float32(-1.0))
            _merge_into_l(l_ref, _sort16_desc(x))
            return 0

        lax.fori_loop(0, (nb + 15) // 16, mbody, 0)
        lv = l_ref[pl.ds(48, 16)]
        fs[0] = lv[15]
        fs[1] = lv[TOPK - 1 - 48]
        tv = jnp.full((16,), fs[0])
        na = ns[3]
        ns[3] = 0
        ns[4] = 0

        def prune_vec(v, iv, rem):
            mval = _iota16() < rem
            m_gt = mval & (v > tv)
            m_eq = mval & (v == tv)
            cs = plsc.cumsum(m_eq.astype(jnp.int32))
            keep_eq = m_eq & ((ns[4] + cs) <= TOPK)
            keep = m_gt | keep_eq
            nn = ns[3]
            plsc.store_compressed(arch_v.at[pl.ds(nn, 16)], v, mask=keep)
            plsc.store_compressed(arch_i.at[pl.ds(nn, 16)], iv, mask=keep)
            ns[3] = nn + _count(keep)
            ns[4] = ns[4] + _count(keep_eq)

        def abody(j, _):
            prune_vec(arch_v[pl.ds(j * 16, 16)], arch_i[pl.ds(j * 16, 16)],
                      na - j * 16)
            return 0

        lax.fori_loop(0, (na + 15) // 16, abody, 0)

        def pbody(j, _):
            prune_vec(buf_v[pl.ds(j * 16, 16)], buf_i[pl.ds(j * 16, 16)],
                      nb - j * 16)
            return 0

        lax.fori_loop(0, (nb + 15) // 16, pbody, 0)
        ns[0] = 0

    def scan_row(a_hbm, row, dst_v, dst_i, do_qat):
        for j in range(4):
            l_ref[pl.ds(j * 16, 16)] = jnp.full((16,), -1.0, jnp.float32)
        ns[0] = 0
        ns[3] = 0
        fs[0] = jnp.float32(-1.0)
        chunk_copy(a_hbm, row, 0, 0).start()
        for c in range(NCHUNK):
            par = c % 2
            chunk_copy(a_hbm, row, c, par).wait()
            if c + 1 < NCHUNK:
                chunk_copy(a_hbm, row, c + 1, (c + 1) % 2).start()
            cb = cbs[par]

            def append_vreg(v, m, idxv):
                n = ns[0]
                plsc.store_compressed(buf_v.at[pl.ds(n, 16)], v, mask=m)
                plsc.store_compressed(buf_i.at[pl.ds(n, 16)], idxv, mask=m)
                ns[0] = n + _count(m)

            if c == 0:
                def sbody(s, _):
                    def vbody(i, _):
                        v = cb[pl.ds((s * SEG + i) * 16, 16)]
                        m = v >= jnp.full((16,), fs[0])
                        append_vreg(
                            v, m,
                            jnp.full((16,), (s * SEG + i) * 16) + _iota16())
                        return 0

                    lax.fori_loop(0, SEG, vbody, 0)
                    pl.when(ns[0] >= BUFCAP)(compact)
                    return 0

                lax.fori_loop(0, NVREG // SEG, sbody, 0)
            else:
                def bbody(b, _):
                    t = jnp.full((16,), fs[0])
                    acc = jnp.zeros((16,), jnp.bool_)
                    for k in range(BLK):
                        v = cb[pl.ds((b * BLK + k) * 16, 16)]
                        acc = acc | (v >= t)

                    def slow():
                        for k in range(BLK):
                            v = cb[pl.ds((b * BLK + k) * 16, 16)]
                            idxv = (jnp.full((16,), c * CHUNK)
                                    + (b * BLK + k) * 16 + _iota16())
                            append_vreg(v, v >= t, idxv)

                    pl.when(_count(acc) > 0)(slow)
                    pl.when(ns[0] >= BUFCAP)(compact)
                    return 0

                lax.fori_loop(0, NBLK, bbody, 0)
            if do_qat:
                base = c * CHUNK
                for j in range(4):
                    spi = sp_i[pl.ds(j * 16, 16)]
                    local = spi - base
                    inr = (local >= 0) & (local < CHUNK)
                    safe = jnp.where(inr, local, 0)
                    g = plsc.load_gather(cb, [safe], mask=inr)
                    cur = qat[pl.ds(j * 16, 16)]
                    qat[pl.ds(j * 16, 16)] = jnp.where(inr, g, cur)
        compact()
        t49 = fs[1]
        t49v = jnp.full((16,), t49)
        na = ns[3]
        ns[1] = 0
        ns[2] = 0

        def selbody(j, _):
            mval = _iota16() < (na - j * 16)
            v = arch_v[pl.ds(j * 16, 16)]
            iv = arch_i[pl.ds(j * 16, 16)]
            m_gt = mval & (v > t49v)
            m_eq = mval & (v == t49v)
            ng = ns[1]
            plsc.store_compressed(gt_v.at[pl.ds(ng, 16)], v, mask=m_gt)
            plsc.store_compressed(gt_i.at[pl.ds(ng, 16)], iv, mask=m_gt)
            ns[1] = ng + _count(m_gt)
            ne = ns[2]
            plsc.store_compressed(eq_i.at[pl.ds(ne, 16)], iv, mask=m_eq)
            ns[2] = ne + _count(m_eq)
            return 0

        lax.fori_loop(0, (na + 15) // 16, selbody, 0)
        g_n = ns[1]
        for j in range(4):
            gl = j * 16 + _iota16()
            in_gt = gl < g_n
            valid = gl < TOPK
            gv = plsc.load_gather(gt_v, [jnp.where(in_gt, gl, 0)], mask=in_gt)
            gi = plsc.load_gather(gt_i, [jnp.where(in_gt, gl, 0)], mask=in_gt)
            ep = jnp.where(in_gt, 0, gl - g_n)
            ei = plsc.load_gather(eq_i, [ep], mask=valid & ~in_gt)
            val = jnp.where(in_gt, gv, jnp.full((16,), t49))
            idx = jnp.where(in_gt, gi, ei)
            dst_v[pl.ds(j * 16, 16)] = jnp.where(valid, val, jnp.float32(0.0))
            dst_i[pl.ds(j * 16, 16)] = jnp.where(valid, idx, -1)

    def row_body(r, _):
        row = wid * ROWS_PER_W + r
        scan_row(p_hbm, row, sp_v, sp_i, do_qat=False)
        for j in range(4):
            qat[pl.ds(j * 16, 16)] = jnp.zeros((16,), jnp.float32)
        scan_row(q_hbm, row, sq_v, sq_i, do_qat=True)
        rbase = row * V
        for j in range(4):
            sqi = sq_i[pl.ds(j * 16, 16)]
            gidx[pl.ds(j * 16, 16)] = jnp.where(sqi >= 0, rbase + sqi, rbase)
        pltpu.make_async_copy(p_hbm.at[gidx], pat, semg).start()
        pltpu.make_async_copy(p_hbm.at[gidx], pat, semg).wait()
        it16 = _iota16()
        for j in range(4):
            sqi = sq_i[pl.ds(j * 16, 16)]
            dup = jnp.zeros((16,), jnp.bool_)

            def dbody(k, dup):
                perm = jnp.where(it16 + k >= 16, it16 + k - 16, it16 + k)
                for jj in range(4):
                    spb = plsc.load_gather(sp_i, [jj * 16 + perm])
                    dup = dup | (sqi == spb)
                return dup

            dup = lax.fori_loop(0, 16, dbody, dup)
            keep = (sqi >= 0) & jnp.logical_not(dup)
            qv = sq_v[pl.ds(j * 16, 16)]
            pv = pat[pl.ds(j * 16, 16)]
            prow[pl.ds(USLOT + j * 16, 16)] = jnp.where(keep, pv, 0.0)
            qrow[pl.ds(USLOT + j * 16, 16)] = jnp.where(keep, qv, 0.0)
            prow[pl.ds(j * 16, 16)] = sp_v[pl.ds(j * 16, 16)]
            qrow[pl.ds(j * 16, 16)] = qat[pl.ds(j * 16, 16)]
        pltpu.sync_copy(prow, pu_hbm.at[row])
        pltpu.sync_copy(qrow, qu_hbm.at[row])
        return 0

    lax.fori_loop(0, ROWS_PER_W, row_body, 0)


def _jsd_tc(pu_ref, qu_ref, out_ref):
    eps = jnp.float32(EPSF)
    pu = pu_ref[...]
    qu = qu_ref[...]
    ps = jnp.sum(pu, axis=1, keepdims=True)
    qs = jnp.sum(qu, axis=1, keepdims=True)
    pn = pu / (ps + eps)
    qn = qu / (qs + eps)
    m = 0.5 * (pn + qn)
    ms = m + eps
    klp = jnp.sum((pn + eps) * jnp.log((pn + eps) / ms), axis=1,
                  keepdims=True)
    klq = jnp.sum((qn + eps) * jnp.log((qn + eps) / ms), axis=1,
                  keepdims=True)
    out_ref[...] = 0.5 * klp + 0.5 * klq


@jax.jit
def kernel(p, q):
    mesh = plsc.VectorSubcoreMesh(
        core_axis_name="c", subcore_axis_name="s",
        num_cores=NCORES, num_subcores=NSUBCORES)
    f32 = jnp.float32
    i32 = jnp.int32
    sc = pl.kernel(
        _sc_body,
        out_type=(
            jax.ShapeDtypeStruct((B, 2 * USLOT), f32),
            jax.ShapeDtypeStruct((B, 2 * USLOT), f32),
        ),
        mesh=mesh,
        scratch_types=[
            pltpu.VMEM((CHUNK,), f32),
            pltpu.VMEM((CHUNK,), f32),
            pltpu.VMEM((64,), f32),
            pltpu.VMEM((BUFSZ,), f32),
            pltpu.VMEM((BUFSZ,), i32),
            pltpu.VMEM((144,), f32),
            pltpu.VMEM((144,), i32),
            pltpu.VMEM((80,), f32),
            pltpu.VMEM((80,), i32),
            pltpu.VMEM((80,), i32),
            pltpu.VMEM((64,), f32),
            pltpu.VMEM((64,), i32),
            pltpu.VMEM((64,), f32),
            pltpu.VMEM((64,), i32),
            pltpu.VMEM((64,), f32),
            pltpu.VMEM((64,), f32),
            pltpu.VMEM((64,), i32),
            pltpu.VMEM((2 * USLOT,), f32),
            pltpu.VMEM((2 * USLOT,), f32),
            pltpu.SMEM((8,), i32),
            pltpu.SMEM((4,), f32),
            pltpu.SemaphoreType.DMA,
            pltpu.SemaphoreType.DMA,
            pltpu.SemaphoreType.DMA,
        ],
        compiler_params=pltpu.CompilerParams(
            use_tc_tiling_on_sc=False, needs_layout_passes=False),
    )
    pu, qu = sc(p.reshape(-1), q.reshape(-1))
    jsd = pl.pallas_call(
        _jsd_tc,
        out_shape=jax.ShapeDtypeStruct((B, 1), f32),
    )(pu, qu)
    return jsd.reshape(B)
